# baseline (speedup 1.0000x reference)
import functools

import jax
import jax.numpy as jnp
from jax import lax
from jax.experimental import pallas as pl
from jax.experimental.pallas import tpu as pltpu
from jax.experimental.pallas import tpu_sc as plsc

_B, _N, _K, _D = 256, 4096, 12, 64
_NW = 32
_BPW = _B // _NW
_G = _N // 16


def _body(slots_hbm, masks_hbm, params_hbm, ps_hbm, pm_hbm,
          mbuf, sbuf, accbuf, pv):
    wid = lax.axis_index("s") * 2 + lax.axis_index("c")

    iota = lax.iota(jnp.int32, 16)
    cols = [jnp.full((16,), k, jnp.int32) for k in range(_K)]
    zero16 = jnp.zeros((16,), jnp.float32)

    pltpu.sync_copy(params_hbm, pv)
    w1v = pv[0, pl.ds(0, 16)]
    b1v = pv[0, pl.ds(16, 16)]
    w2v = pv[0, pl.ds(32, 16)]
    w1bc = [w1v[i16] for i16 in range(16)]
    b1bc = [b1v[i16] for i16 in range(16)]
    w2bc = [w2v[i16] for i16 in range(16)]
    b2bc = jnp.zeros((16,), jnp.float32) + pv[0, pl.ds(48, 16)][0]

    for k in range(_K, 16):
        accbuf[k, :] = zero16

    for i in range(_BPW):
        b = wid * _BPW + i
        pltpu.sync_copy(masks_hbm.at[b], mbuf)
        pltpu.sync_copy(slots_hbm.at[b], sbuf)

        def p1(t, accs):
            rows = t * 16 + iota
            return tuple(accs[k] + plsc.load_gather(mbuf, [rows, cols[k]])
                         for k in range(_K))
        accs = lax.fori_loop(0, _G, p1, tuple(zero16 for _ in range(_K)))
        for k in range(_K):
            accbuf[k, :] = accs[k]
        uv = zero16
        for i16 in range(16):
            uv = uv + plsc.load_gather(
                accbuf, [iota, jnp.full((16,), i16, jnp.int32)])
        util = uv * (1.0 / _N)

        acc = b2bc
        for h in range(16):
            hh = jnp.maximum(util * w1bc[h] + b1bc[h], 0.0)
            acc = acc + hh * w2bc[h]
        g = 1.0 / (1.0 + jnp.exp(-acc))
        gbc = [g[k] for k in range(_K)]

        def p2(t, carry):
            rows = t * 16 + iota
            vals = [plsc.load_gather(mbuf, [rows, cols[k]]) * gbc[k]
                    for k in range(_K)]
            s01 = (vals[0] + vals[1]) + (vals[2] + vals[3])
            s23 = (vals[4] + vals[5]) + (vals[6] + vals[7])
            s45 = (vals[8] + vals[9]) + (vals[10] + vals[11])
            r = 1.0 / ((s01 + s23) + s45 + 1e-8)
            for k in range(_K):
                plsc.store_scatter(mbuf, [rows, cols[k]], vals[k] * r)
            return carry
        lax.fori_loop(0, _G, p2, 0)
        pltpu.sync_copy(mbuf, pm_hbm.at[b])

        for k in range(_K):
            for c in range(_D // 16):
                sl = sbuf[k, pl.ds(c * 16, 16)]
                sbuf[k, pl.ds(c * 16, 16)] = sl * gbc[k]
        pltpu.sync_copy(sbuf, ps_hbm.at[b])


def kernel(slots, masks, w1, b1, w2, b2):
    params = jnp.zeros((64,), jnp.float32)
    params = params.at[0:16].set(w1.reshape(-1))
    params = params.at[16:32].set(b1.reshape(-1))
    params = params.at[32:48].set(w2.reshape(-1))
    params = params.at[48].set(b2.reshape(-1)[0]).reshape(1, 64)

    mesh = plsc.VectorSubcoreMesh(core_axis_name="c", subcore_axis_name="s",
                                  num_cores=2, num_subcores=16)
    run = functools.partial(
        pl.kernel,
        out_type=[
            jax.ShapeDtypeStruct((_B, _K, _D), jnp.float32),
            jax.ShapeDtypeStruct((_B, _N, _K), jnp.float32),
        ],
        mesh=mesh,
        compiler_params=pltpu.CompilerParams(
            needs_layout_passes=False, use_tc_tiling_on_sc=False),
        scratch_types=[
            pltpu.VMEM((_N, _K), jnp.float32),
            pltpu.VMEM((_K, _D), jnp.float32),
            pltpu.VMEM((16, 16), jnp.float32),
            pltpu.VMEM((1, 64), jnp.float32),
        ],
    )(_body)
    ps, pm = run(slots, masks, params)
    return (ps, pm)


# speedup vs baseline: 4.6002x; 4.6002x over previous
import functools

import jax
import jax.numpy as jnp
from jax import lax
from jax.experimental import pallas as pl
from jax.experimental.pallas import tpu as pltpu
from jax.experimental.pallas import tpu_sc as plsc

_B, _N, _K, _D = 256, 4096, 12, 64
_NW = 32
_BPW = _B // _NW
_G = _N // 16


def _sc_body(masksT_hbm, params_hbm, pmT_hbm, gates_hbm,
             mbuf, accbuf, gbuf, pv):
    wid = lax.axis_index("s") * 2 + lax.axis_index("c")

    iota = lax.iota(jnp.int32, 16)
    zero16 = jnp.zeros((16,), jnp.float32)

    pltpu.sync_copy(params_hbm, pv)
    w1v = pv[0, pl.ds(0, 16)]
    b1v = pv[0, pl.ds(16, 16)]
    w2v = pv[0, pl.ds(32, 16)]
    w1bc = [w1v[i16] for i16 in range(16)]
    b1bc = [b1v[i16] for i16 in range(16)]
    w2bc = [w2v[i16] for i16 in range(16)]
    b2bc = jnp.zeros((16,), jnp.float32) + pv[0, pl.ds(48, 16)][0]

    for k in range(_K, 16):
        accbuf[k, :] = zero16

    for i in range(_BPW):
        b = wid * _BPW + i
        for k in range(_K):
            pltpu.sync_copy(masksT_hbm.at[k, b, :], mbuf.at[k, :])

        def p1(t, accs):
            sl = pl.ds(t * 16, 16)
            return tuple(accs[k] + mbuf[k, sl] for k in range(_K))
        accs = lax.fori_loop(0, _G, p1, tuple(zero16 for _ in range(_K)))
        for k in range(_K):
            accbuf[k, :] = accs[k]
        uv = zero16
        for i16 in range(16):
            uv = uv + plsc.load_gather(
                accbuf, [iota, jnp.full((16,), i16, jnp.int32)])
        util = uv * (1.0 / _N)

        acc = b2bc
        for h in range(16):
            hh = jnp.maximum(util * w1bc[h] + b1bc[h], 0.0)
            acc = acc + hh * w2bc[h]
        g = 1.0 / (1.0 + jnp.exp(-acc))
        gbuf[...] = g
        pltpu.sync_copy(gbuf, gates_hbm.at[b])
        gbc = [g[k] for k in range(_K)]

        def p2(t, carry):
            sl = pl.ds(t * 16, 16)
            vals = [mbuf[k, sl] * gbc[k] for k in range(_K)]
            s01 = (vals[0] + vals[1]) + (vals[2] + vals[3])
            s23 = (vals[4] + vals[5]) + (vals[6] + vals[7])
            s45 = (vals[8] + vals[9]) + (vals[10] + vals[11])
            r = 1.0 / ((s01 + s23) + s45 + 1e-8)
            for k in range(_K):
                mbuf[k, sl] = vals[k] * r
            return carry
        lax.fori_loop(0, _G, p2, 0)
        for k in range(_K):
            pltpu.sync_copy(mbuf.at[k, :], pmT_hbm.at[k, b, :])


def _tc_slots_body(slots_ref, gates_ref, ps_ref):
    g = gates_ref[...][:, :_K]
    ps_ref[...] = slots_ref[...] * g[:, :, None]


def kernel(slots, masks, w1, b1, w2, b2):
    params = jnp.zeros((64,), jnp.float32)
    params = params.at[0:16].set(w1.reshape(-1))
    params = params.at[16:32].set(b1.reshape(-1))
    params = params.at[32:48].set(w2.reshape(-1))
    params = params.at[48].set(b2.reshape(-1)[0]).reshape(1, 64)

    masksT = jnp.transpose(masks, (2, 0, 1))

    mesh = plsc.VectorSubcoreMesh(core_axis_name="c", subcore_axis_name="s",
                                  num_cores=2, num_subcores=16)
    run = functools.partial(
        pl.kernel,
        out_type=[
            jax.ShapeDtypeStruct((_K, _B, _N), jnp.float32),
            jax.ShapeDtypeStruct((_B, 16), jnp.float32),
        ],
        mesh=mesh,
        compiler_params=pltpu.CompilerParams(
            needs_layout_passes=False, use_tc_tiling_on_sc=False),
        scratch_types=[
            pltpu.VMEM((_K, _N), jnp.float32),
            pltpu.VMEM((16, 16), jnp.float32),
            pltpu.VMEM((16,), jnp.float32),
            pltpu.VMEM((1, 64), jnp.float32),
        ],
    )(_sc_body)
    pmT, gates = run(masksT, params)

    ps = pl.pallas_call(
        _tc_slots_body,
        out_shape=jax.ShapeDtypeStruct((_B, _K, _D), jnp.float32),
    )(slots, gates)
    return (ps, jnp.transpose(pmT, (1, 2, 0)))


# speedup vs baseline: 7.4278x; 1.6147x over previous
import functools

import jax
import jax.numpy as jnp
from jax import lax
from jax.experimental import pallas as pl
from jax.experimental.pallas import tpu as pltpu
from jax.experimental.pallas import tpu_sc as plsc

_B, _N, _K, _D = 256, 4096, 12, 64
_NW = 32
_BPW = _B // _NW
_G = _N // 16


def _sc_body(masksT_hbm, params_hbm, pmT_hbm, gates_hbm,
             mbuf, accbuf, gbuf, pv, lsem0, lsem1, ssem0, ssem1):
    wid = lax.axis_index("s") * 2 + lax.axis_index("c")

    iota = lax.iota(jnp.int32, 16)
    zero16 = jnp.zeros((16,), jnp.float32)

    pltpu.sync_copy(params_hbm, pv)
    w1v = pv[0, pl.ds(0, 16)]
    b1v = pv[0, pl.ds(16, 16)]
    w2v = pv[0, pl.ds(32, 16)]
    w1bc = [w1v[i16] for i16 in range(16)]
    b1bc = [b1v[i16] for i16 in range(16)]
    w2bc = [w2v[i16] for i16 in range(16)]
    b2bc = jnp.zeros((16,), jnp.float32) + pv[0, pl.ds(48, 16)][0]

    for k in range(_K, 16):
        accbuf[k, :] = zero16

    lsem = [lsem0, lsem1]
    ssem = [ssem0, ssem1]

    def start_load(i):
        b = wid * _BPW + i
        p = i % 2
        return [pltpu.async_copy(masksT_hbm.at[k, b, :], mbuf.at[p, k, :],
                                 lsem[p]) for k in range(_K)]

    def start_store(i):
        b = wid * _BPW + i
        p = i % 2
        return [pltpu.async_copy(mbuf.at[p, k, :], pmT_hbm.at[k, b, :],
                                 ssem[p]) for k in range(_K)]

    loads = {0: start_load(0)}
    stores = {}
    for i in range(_BPW):
        b = wid * _BPW + i
        p = i % 2
        if i + 1 < _BPW:
            if i - 1 >= 0:
                for dsc in stores[i - 1]:
                    dsc.wait()
            loads[i + 1] = start_load(i + 1)
        for dsc in loads[i]:
            dsc.wait()

        def p1(t, accs):
            sl = pl.ds(t * 16, 16)
            return tuple(accs[k] + mbuf[p, k, sl] for k in range(_K))
        accs = lax.fori_loop(0, _G, p1, tuple(zero16 for _ in range(_K)))
        for k in range(_K):
            accbuf[k, :] = accs[k]
        uv = zero16
        for i16 in range(16):
            uv = uv + plsc.load_gather(
                accbuf, [iota, jnp.full((16,), i16, jnp.int32)])
        util = uv * (1.0 / _N)

        acc = b2bc
        for h in range(16):
            hh = jnp.maximum(util * w1bc[h] + b1bc[h], 0.0)
            acc = acc + hh * w2bc[h]
        g = 1.0 / (1.0 + jnp.exp(-acc))
        gbuf[...] = g
        pltpu.sync_copy(gbuf, gates_hbm.at[b])
        gbc = [g[k] for k in range(_K)]

        @plsc.parallel_loop(0, _G, 1, unroll=2)
        def p2(t):
            sl = pl.ds(t * 16, 16)
            vals = [mbuf[p, k, sl] * gbc[k] for k in range(_K)]
            s01 = (vals[0] + vals[1]) + (vals[2] + vals[3])
            s23 = (vals[4] + vals[5]) + (vals[6] + vals[7])
            s45 = (vals[8] + vals[9]) + (vals[10] + vals[11])
            r = 1.0 / ((s01 + s23) + s45 + 1e-8)
            for k in range(_K):
                mbuf[p, k, sl] = vals[k] * r
        stores[i] = start_store(i)
    for i in (_BPW - 2, _BPW - 1):
        for dsc in stores[i]:
            dsc.wait()


def _tc_slots_body(slots_ref, gates_ref, ps_ref):
    g = gates_ref[...][:, :_K]
    ps_ref[...] = slots_ref[...] * g[:, :, None]


def kernel(slots, masks, w1, b1, w2, b2):
    params = jnp.zeros((64,), jnp.float32)
    params = params.at[0:16].set(w1.reshape(-1))
    params = params.at[16:32].set(b1.reshape(-1))
    params = params.at[32:48].set(w2.reshape(-1))
    params = params.at[48].set(b2.reshape(-1)[0]).reshape(1, 64)

    masksT = jnp.transpose(masks, (2, 0, 1))

    mesh = plsc.VectorSubcoreMesh(core_axis_name="c", subcore_axis_name="s",
                                  num_cores=2, num_subcores=16)
    run = functools.partial(
        pl.kernel,
        out_type=[
            jax.ShapeDtypeStruct((_K, _B, _N), jnp.float32),
            jax.ShapeDtypeStruct((_B, 16), jnp.float32),
        ],
        mesh=mesh,
        compiler_params=pltpu.CompilerParams(
            needs_layout_passes=False, use_tc_tiling_on_sc=False),
        scratch_types=[
            pltpu.VMEM((2, _K, _N), jnp.float32),
            pltpu.VMEM((16, 16), jnp.float32),
            pltpu.VMEM((16,), jnp.float32),
            pltpu.VMEM((1, 64), jnp.float32),
            pltpu.SemaphoreType.DMA,
            pltpu.SemaphoreType.DMA,
            pltpu.SemaphoreType.DMA,
            pltpu.SemaphoreType.DMA,
        ],
    )(_sc_body)
    pmT, gates = run(masksT, params)

    ps = pl.pallas_call(
        _tc_slots_body,
        out_shape=jax.ShapeDtypeStruct((_B, _K, _D), jnp.float32),
    )(slots, gates)
    return (ps, jnp.transpose(pmT, (1, 2, 0)))


# speedup vs baseline: 10.0771x; 1.3567x over previous
import functools

import jax
import jax.numpy as jnp
from jax import lax
from jax.experimental import pallas as pl
from jax.experimental.pallas import tpu as pltpu
from jax.experimental.pallas import tpu_sc as plsc

_B, _N, _K, _D = 256, 4096, 12, 64
_NW = 32
_BPW = _B // _NW
_G = _N // 16


def _sc_body(masksT_hbm, params_hbm, pmT_hbm, gates_hbm,
             mbuf, accbuf, gbuf, pv, lsem0, lsem1, ssem0, ssem1):
    wid = lax.axis_index("s") * 2 + lax.axis_index("c")

    iota = lax.iota(jnp.int32, 16)
    zero16 = jnp.zeros((16,), jnp.float32)

    pltpu.sync_copy(params_hbm, pv)
    w1v = pv[0, pl.ds(0, 16)]
    b1v = pv[0, pl.ds(16, 16)]
    w2v = pv[0, pl.ds(32, 16)]
    w1bc = [w1v[i16] for i16 in range(16)]
    b1bc = [b1v[i16] for i16 in range(16)]
    w2bc = [w2v[i16] for i16 in range(16)]
    b2bc = jnp.zeros((16,), jnp.float32) + pv[0, pl.ds(48, 16)][0]

    for k in range(_K, 16):
        accbuf[k, :] = zero16

    lsem = [lsem0, lsem1]
    ssem = [ssem0, ssem1]

    def start_load(i):
        b = wid * _BPW + i
        p = i % 2
        return [pltpu.async_copy(masksT_hbm.at[k, b, :],
                                 mbuf.at[p * _K + k, :],
                                 lsem[p]) for k in range(_K)]

    def start_store(i):
        b = wid * _BPW + i
        p = i % 2
        return [pltpu.async_copy(mbuf.at[p * _K + k, :], pmT_hbm.at[k, b, :],
                                 ssem[p]) for k in range(_K)]

    loads = {0: start_load(0)}
    stores = {}
    for i in range(_BPW):
        b = wid * _BPW + i
        p = i % 2
        if i + 1 < _BPW:
            if i - 1 >= 0:
                for dsc in stores[i - 1]:
                    dsc.wait()
            loads[i + 1] = start_load(i + 1)
        for dsc in loads[i]:
            dsc.wait()

        def p1(t, accs):
            sl = pl.ds(t * 16, 16)
            return tuple(accs[k] + mbuf[p * _K + k, sl] for k in range(_K))
        accs = lax.fori_loop(0, _G, p1, tuple(zero16 for _ in range(_K)))
        for k in range(_K):
            accbuf[k, :] = accs[k]
        uv = zero16
        for i16 in range(16):
            uv = uv + plsc.load_gather(
                accbuf, [iota, jnp.full((16,), i16, jnp.int32)])
        util = uv * (1.0 / _N)

        acc = b2bc
        for h in range(16):
            hh = jnp.maximum(util * w1bc[h] + b1bc[h], 0.0)
            acc = acc + hh * w2bc[h]
        g = 1.0 / (1.0 + jnp.exp(-acc))
        gbuf[...] = g
        pltpu.sync_copy(gbuf, gates_hbm.at[b])
        gbc = [g[k] for k in range(_K)]

        @plsc.parallel_loop(0, _G, 1, unroll=2)
        def p2(t):
            sl = pl.ds(t * 16, 16)
            vals = [mbuf[p * _K + k, sl] * gbc[k] for k in range(_K)]
            s01 = (vals[0] + vals[1]) + (vals[2] + vals[3])
            s23 = (vals[4] + vals[5]) + (vals[6] + vals[7])
            s45 = (vals[8] + vals[9]) + (vals[10] + vals[11])
            r = 1.0 / ((s01 + s23) + s45 + 1e-8)
            for k in range(_K):
                mbuf[p * _K + k, sl] = vals[k] * r
        stores[i] = start_store(i)
    for i in (_BPW - 2, _BPW - 1):
        for dsc in stores[i]:
            dsc.wait()


def _tc_slots_body(slots_ref, gates_ref, ps_ref):
    g = gates_ref[...][:, :_K]
    ps_ref[...] = slots_ref[...] * g[:, :, None]


def kernel(slots, masks, w1, b1, w2, b2):
    params = jnp.zeros((64,), jnp.float32)
    params = params.at[0:16].set(w1.reshape(-1))
    params = params.at[16:32].set(b1.reshape(-1))
    params = params.at[32:48].set(w2.reshape(-1))
    params = params.at[48].set(b2.reshape(-1)[0]).reshape(1, 64)

    masksT = jnp.transpose(masks, (2, 0, 1))

    mesh = plsc.VectorSubcoreMesh(core_axis_name="c", subcore_axis_name="s",
                                  num_cores=2, num_subcores=16)
    run = functools.partial(
        pl.kernel,
        out_type=[
            jax.ShapeDtypeStruct((_K, _B, _N), jnp.float32),
            jax.ShapeDtypeStruct((_B, 16), jnp.float32),
        ],
        mesh=mesh,
        compiler_params=pltpu.CompilerParams(
            needs_layout_passes=False, use_tc_tiling_on_sc=True),
        scratch_types=[
            pltpu.VMEM((2 * _K, _N), jnp.float32),
            pltpu.VMEM((16, 16), jnp.float32),
            pltpu.VMEM((16,), jnp.float32),
            pltpu.VMEM((1, 64), jnp.float32),
            pltpu.SemaphoreType.DMA,
            pltpu.SemaphoreType.DMA,
            pltpu.SemaphoreType.DMA,
            pltpu.SemaphoreType.DMA,
        ],
    )(_sc_body)
    pmT, gates = run(masksT, params)

    ps = pl.pallas_call(
        _tc_slots_body,
        out_shape=jax.ShapeDtypeStruct((_B, _K, _D), jnp.float32),
    )(slots, gates)
    return (ps, jnp.transpose(pmT, (1, 2, 0)))


# speedup vs baseline: 11.7946x; 1.1704x over previous
import functools

import jax
import jax.numpy as jnp
from jax import lax
from jax.experimental import pallas as pl
from jax.experimental.pallas import tpu as pltpu
from jax.experimental.pallas import tpu_sc as plsc

_B, _N, _K, _D = 256, 4096, 12, 64
_NW = 32
_BPW = _B // _NW
_G = _N // 16


def _sc_body(masksT_hbm, params_hbm, pmT_hbm, gates_hbm,
             mbuf, accbuf, gbuf, pv, lsem0, lsem1, ssem0, ssem1):
    wid = lax.axis_index("s") * 2 + lax.axis_index("c")

    iota = lax.iota(jnp.int32, 16)
    zero16 = jnp.zeros((16,), jnp.float32)

    pltpu.sync_copy(params_hbm, pv)
    w1v = pv[0, pl.ds(0, 16)]
    b1v = pv[0, pl.ds(16, 16)]
    w2v = pv[0, pl.ds(32, 16)]
    w1bc = [w1v[i16] for i16 in range(16)]
    b1bc = [b1v[i16] for i16 in range(16)]
    w2bc = [w2v[i16] for i16 in range(16)]
    b2bc = jnp.zeros((16,), jnp.float32) + pv[0, pl.ds(48, 16)][0]

    for k in range(_K, 16):
        accbuf[k, :] = zero16

    lsem = [lsem0, lsem1]
    ssem = [ssem0, ssem1]

    def start_load(i):
        b = wid * _BPW + i
        p = i % 2
        return [pltpu.async_copy(masksT_hbm.at[k, b, :],
                                 mbuf.at[p * _K + k, :],
                                 lsem[p]) for k in range(_K)]

    def start_store(i):
        b = wid * _BPW + i
        p = i % 2
        return [pltpu.async_copy(mbuf.at[p * _K + k, :], pmT_hbm.at[k, b, :],
                                 ssem[p]) for k in range(_K)]

    loads = {0: start_load(0)}
    stores = {}
    for i in range(_BPW):
        b = wid * _BPW + i
        p = i % 2
        if i + 1 < _BPW:
            if i - 1 >= 0:
                for dsc in stores[i - 1]:
                    dsc.wait()
            loads[i + 1] = start_load(i + 1)
        for dsc in loads[i]:
            dsc.wait()

        def p1(t2, accs):
            out = list(accs)
            for s in range(8):
                sl = pl.ds(t2 * 128 + s * 16, 16)
                for k in range(_K):
                    out[k] = out[k] + mbuf[p * _K + k, sl]
            return tuple(out)
        accs = lax.fori_loop(0, _N // 128, p1,
                             tuple(zero16 for _ in range(_K)))
        for k in range(_K):
            accbuf[k, :] = accs[k]
        uv = zero16
        for i16 in range(16):
            uv = uv + plsc.load_gather(
                accbuf, [iota, jnp.full((16,), i16, jnp.int32)])
        util = uv * (1.0 / _N)

        acc = b2bc
        for h in range(16):
            hh = jnp.maximum(util * w1bc[h] + b1bc[h], 0.0)
            acc = acc + hh * w2bc[h]
        g = 1.0 / (1.0 + jnp.exp(-acc))
        gbuf[...] = g
        pltpu.sync_copy(gbuf, gates_hbm.at[b])
        gbc = [g[k] for k in range(_K)]

        def p2(t2, carry):
            for s in range(8):
                sl = pl.ds(t2 * 128 + s * 16, 16)
                vals = [mbuf[p * _K + k, sl] * gbc[k] for k in range(_K)]
                s01 = (vals[0] + vals[1]) + (vals[2] + vals[3])
                s23 = (vals[4] + vals[5]) + (vals[6] + vals[7])
                s45 = (vals[8] + vals[9]) + (vals[10] + vals[11])
                r = 1.0 / ((s01 + s23) + s45 + 1e-8)
                for k in range(_K):
                    mbuf[p * _K + k, sl] = vals[k] * r
            return carry
        lax.fori_loop(0, _N // 128, p2, 0)
        stores[i] = start_store(i)
    for i in (_BPW - 2, _BPW - 1):
        for dsc in stores[i]:
            dsc.wait()


def _tc_slots_body(slots_ref, gates_ref, ps_ref):
    g = gates_ref[...][:, :_K]
    ps_ref[...] = slots_ref[...] * g[:, :, None]


def kernel(slots, masks, w1, b1, w2, b2):
    params = jnp.zeros((64,), jnp.float32)
    params = params.at[0:16].set(w1.reshape(-1))
    params = params.at[16:32].set(b1.reshape(-1))
    params = params.at[32:48].set(w2.reshape(-1))
    params = params.at[48].set(b2.reshape(-1)[0]).reshape(1, 64)

    masksT = jnp.transpose(masks, (2, 0, 1))

    mesh = plsc.VectorSubcoreMesh(core_axis_name="c", subcore_axis_name="s",
                                  num_cores=2, num_subcores=16)
    run = functools.partial(
        pl.kernel,
        out_type=[
            jax.ShapeDtypeStruct((_K, _B, _N), jnp.float32),
            jax.ShapeDtypeStruct((_B, 16), jnp.float32),
        ],
        mesh=mesh,
        compiler_params=pltpu.CompilerParams(
            needs_layout_passes=False, use_tc_tiling_on_sc=True),
        scratch_types=[
            pltpu.VMEM((2 * _K, _N), jnp.float32),
            pltpu.VMEM((16, 16), jnp.float32),
            pltpu.VMEM((16,), jnp.float32),
            pltpu.VMEM((1, 64), jnp.float32),
            pltpu.SemaphoreType.DMA,
            pltpu.SemaphoreType.DMA,
            pltpu.SemaphoreType.DMA,
            pltpu.SemaphoreType.DMA,
        ],
    )(_sc_body)
    pmT, gates = run(masksT, params)

    ps = pl.pallas_call(
        _tc_slots_body,
        out_shape=jax.ShapeDtypeStruct((_B, _K, _D), jnp.float32),
    )(slots, gates)
    return (ps, jnp.transpose(pmT, (1, 2, 0)))


# speedup vs baseline: 11.8206x; 1.0022x over previous
import functools

import jax
import jax.numpy as jnp
from jax import lax
from jax.experimental import pallas as pl
from jax.experimental.pallas import tpu as pltpu
from jax.experimental.pallas import tpu_sc as plsc

_B, _N, _K, _D = 256, 4096, 12, 64
_NW = 32
_BPW = _B // _NW
_G = _N // 16


def _sc_body(masksT_hbm, params_hbm, pmT_hbm, gates_hbm,
             mbuf, accbuf, gbuf, pv, lsem0, lsem1, ssem0, ssem1):
    wid = lax.axis_index("s") * 2 + lax.axis_index("c")

    iota = lax.iota(jnp.int32, 16)
    zero16 = jnp.zeros((16,), jnp.float32)

    pltpu.sync_copy(params_hbm, pv)
    w1v = pv[0, pl.ds(0, 16)]
    b1v = pv[0, pl.ds(16, 16)]
    w2v = pv[0, pl.ds(32, 16)]
    w1bc = [w1v[i16] for i16 in range(16)]
    b1bc = [b1v[i16] for i16 in range(16)]
    w2bc = [w2v[i16] for i16 in range(16)]
    b2bc = jnp.zeros((16,), jnp.float32) + pv[0, pl.ds(48, 16)][0]

    for k in range(_K, 16):
        accbuf[k, :] = zero16

    lsem = [lsem0, lsem1]
    ssem = [ssem0, ssem1]

    def start_load(i):
        b = wid * _BPW + i
        p = i % 2
        return [pltpu.async_copy(masksT_hbm.at[k, b, :],
                                 mbuf.at[p * _K + k, :],
                                 lsem[p]) for k in range(_K)]

    def start_store(i):
        b = wid * _BPW + i
        p = i % 2
        return [pltpu.async_copy(mbuf.at[p * _K + k, :], pmT_hbm.at[k, b, :],
                                 ssem[p]) for k in range(_K)]

    loads = {0: start_load(0)}
    stores = {}
    for i in range(_BPW):
        b = wid * _BPW + i
        p = i % 2
        if i + 1 < _BPW:
            if i - 1 >= 0:
                for dsc in stores[i - 1]:
                    dsc.wait()
            loads[i + 1] = start_load(i + 1)
        for dsc in loads[i]:
            dsc.wait()

        def p1(t2, accs):
            out = list(accs)
            for s in range(8):
                sl = pl.ds(t2 * 128 + s * 16, 16)
                for k in range(_K):
                    out[k] = out[k] + mbuf[p * _K + k, sl]
            return tuple(out)
        accs = lax.fori_loop(0, _N // 128, p1,
                             tuple(zero16 for _ in range(_K)))
        for k in range(_K):
            accbuf[k, :] = accs[k]
        uv = zero16
        for i16 in range(16):
            uv = uv + plsc.load_gather(
                accbuf, [iota, jnp.full((16,), i16, jnp.int32)])
        util = uv * (1.0 / _N)

        acc = b2bc
        for h in range(16):
            hh = jnp.maximum(util * w1bc[h] + b1bc[h], 0.0)
            acc = acc + hh * w2bc[h]
        g = 1.0 / (1.0 + jnp.exp(-acc))
        gbuf[i, :] = g
        gbc = [g[k] for k in range(_K)]

        def p2(t2, carry):
            for s in range(8):
                sl = pl.ds(t2 * 128 + s * 16, 16)
                vals = [mbuf[p * _K + k, sl] * gbc[k] for k in range(_K)]
                s01 = (vals[0] + vals[1]) + (vals[2] + vals[3])
                s23 = (vals[4] + vals[5]) + (vals[6] + vals[7])
                s45 = (vals[8] + vals[9]) + (vals[10] + vals[11])
                r = 1.0 / ((s01 + s23) + s45 + 1e-8)
                for k in range(_K):
                    mbuf[p * _K + k, sl] = vals[k] * r
            return carry
        lax.fori_loop(0, _N // 128, p2, 0)
        stores[i] = start_store(i)
    pltpu.sync_copy(gbuf, gates_hbm.at[pl.ds(wid * _BPW, _BPW), :])
    for i in (_BPW - 2, _BPW - 1):
        for dsc in stores[i]:
            dsc.wait()


def _tc_slots_body(slots_ref, gates_ref, ps_ref):
    g = gates_ref[...][:, :_K]
    ps_ref[...] = slots_ref[...] * g[:, :, None]


def kernel(slots, masks, w1, b1, w2, b2):
    params = jnp.concatenate(
        [w1.reshape(-1), b1.reshape(-1), w2.reshape(-1), b2.reshape(-1),
         jnp.zeros((15,), jnp.float32)]).reshape(1, 64)

    masksT = jnp.transpose(masks, (2, 0, 1))

    mesh = plsc.VectorSubcoreMesh(core_axis_name="c", subcore_axis_name="s",
                                  num_cores=2, num_subcores=16)
    run = functools.partial(
        pl.kernel,
        out_type=[
            jax.ShapeDtypeStruct((_K, _B, _N), jnp.float32),
            jax.ShapeDtypeStruct((_B, 16), jnp.float32),
        ],
        mesh=mesh,
        compiler_params=pltpu.CompilerParams(
            needs_layout_passes=False, use_tc_tiling_on_sc=True),
        scratch_types=[
            pltpu.VMEM((2 * _K, _N), jnp.float32),
            pltpu.VMEM((16, 16), jnp.float32),
            pltpu.VMEM((_BPW, 16), jnp.float32),
            pltpu.VMEM((1, 64), jnp.float32),
            pltpu.SemaphoreType.DMA,
            pltpu.SemaphoreType.DMA,
            pltpu.SemaphoreType.DMA,
            pltpu.SemaphoreType.DMA,
        ],
    )(_sc_body)
    pmT, gates = run(masksT, params)

    ps = pl.pallas_call(
        _tc_slots_body,
        out_shape=jax.ShapeDtypeStruct((_B, _K, _D), jnp.float32),
    )(slots, gates)
    return (ps, jnp.transpose(pmT, (1, 2, 0)))
